# Initial kernel scaffold; baseline (speedup 1.0000x reference)
import jax
import jax.numpy as jnp
from jax.experimental import pallas as pl


def kernel(feat, edge_index, edge_noise, W, b):
    raise NotImplementedError("write your pallas kernel here")


# speedup vs baseline: 3.7791x; 3.7791x over previous
import functools

import jax
import jax.numpy as jnp
from jax import lax
from jax.experimental import pallas as pl
from jax.experimental.pallas import tpu as pltpu
from jax.experimental.pallas import tpu_sc as plsc

N = 10000
E = 320000
D = 128
L = 16
NC = 2
NS = 16
NW = NC * NS
NPAD = 10240

E_PER_TILE_DEG = E // NS
E_PER_TILE = E // NW
CHUNK = 80
NCHUNK = E_PER_TILE // CHUNK
ROWS_PER_TILE = NPAD // NS
ZROWS = 128

_mesh = plsc.VectorSubcoreMesh(core_axis_name="c", subcore_axis_name="s")
_sc_params = pltpu.CompilerParams(needs_layout_passes=False)


@functools.partial(
    pl.kernel,
    out_type=jax.ShapeDtypeStruct((2 * NPAD,), jnp.float32),
    mesh=_mesh,
    scratch_types=[
        pltpu.VMEM((E_PER_TILE_DEG,), jnp.int32),
        pltpu.VMEM((NPAD,), jnp.float32),
        pltpu.VMEM((NPAD // NS,), jnp.float32),
        pltpu.VMEM((NPAD // NS,), jnp.float32),
        pltpu.VMEM_SHARED((NS, NPAD), jnp.float32),
    ],
    compiler_params=_sc_params,
)
def _degrees(src_hbm, dst_hbm, out_hbm, idx_v, acc, tbuf, sbuf, shared):
    c = lax.axis_index("c")
    s = lax.axis_index("s")
    base = s * E_PER_TILE_DEG

    @pl.when(c == 0)
    def _():
        pltpu.sync_copy(src_hbm.at[pl.ds(base, E_PER_TILE_DEG)], idx_v)

    @pl.when(c == 1)
    def _():
        pltpu.sync_copy(dst_hbm.at[pl.ds(base, E_PER_TILE_DEG)], idx_v)

    zeros16 = jnp.zeros((L,), jnp.float32)
    ones16 = jnp.ones((L,), jnp.float32)

    def zero_body(i, _):
        acc[pl.ds(i * L, L)] = zeros16
        return 0

    lax.fori_loop(0, NPAD // L, zero_body, 0)

    def count_body(i, _):
        iv = idx_v[pl.ds(i * L, L)]
        plsc.addupdate_scatter(acc, [iv], ones16)
        return 0

    lax.fori_loop(0, E_PER_TILE_DEG // L, count_body, 0)

    pltpu.sync_copy(acc, shared.at[s])
    plsc.subcore_barrier()

    seg = NPAD // NS
    rbase = s * seg

    def zred(i, _):
        sbuf[pl.ds(i * L, L)] = zeros16
        return 0

    lax.fori_loop(0, seg // L, zred, 0)

    for t in range(NS):
        pltpu.sync_copy(shared.at[t, pl.ds(rbase, seg)], tbuf)

        def radd(i, _):
            sbuf[pl.ds(i * L, L)] = sbuf[pl.ds(i * L, L)] + tbuf[pl.ds(i * L, L)]
            return 0

        lax.fori_loop(0, seg // L, radd, 0)

    pltpu.sync_copy(sbuf, out_hbm.at[pl.ds(c * NPAD + rbase, seg)])


@functools.partial(
    pl.kernel,
    out_type=jax.ShapeDtypeStruct((NC, NPAD, D), jnp.float32),
    mesh=_mesh,
    scratch_types=[
        pltpu.VMEM((CHUNK,), jnp.int32),
        pltpu.VMEM((CHUNK,), jnp.int32),
        pltpu.VMEM((CHUNK, D), jnp.float32),
        pltpu.VMEM((CHUNK, D), jnp.float32),
        pltpu.VMEM((CHUNK, D), jnp.float32),
        pltpu.VMEM((ZROWS, D), jnp.float32),
        pltpu.VMEM_SHARED((NPAD, D), jnp.float32),
        pltpu.SemaphoreType.DMA,
        pltpu.SemaphoreType.DMA,
    ],
    compiler_params=_sc_params,
)
def _aggregate(h_hbm, src_hbm, dst_hbm, noise_hbm, out_hbm,
               sidx, didx, rows, nbuf, mbuf, zbuf, agg, sem_g, sem_n):
    c = lax.axis_index("c")
    s = lax.axis_index("s")
    wid = c * NS + s
    ebase = wid * E_PER_TILE

    zeros16 = jnp.zeros((L,), jnp.float32)

    def zrow(r, _):
        def zcol(q, _):
            zbuf[r, pl.ds(q * L, L)] = zeros16
            return 0

        lax.fori_loop(0, D // L, zcol, 0, unroll=True)
        return 0

    lax.fori_loop(0, ZROWS, zrow, 0)

    abase = s * ROWS_PER_TILE
    for r in range(ROWS_PER_TILE // ZROWS):
        pltpu.sync_copy(zbuf, agg.at[pl.ds(abase + r * ZROWS, ZROWS), :])
    plsc.subcore_barrier()

    def chunk_body(k, _):
        b = ebase + k * CHUNK
        pltpu.sync_copy(src_hbm.at[pl.ds(b, CHUNK)], sidx)
        pltpu.sync_copy(dst_hbm.at[pl.ds(b, CHUNK)], didx)
        g = pltpu.async_copy(h_hbm.at[sidx], rows, sem_g)
        n = pltpu.async_copy(noise_hbm.at[pl.ds(b, CHUNK), :], nbuf, sem_n)
        g.wait()
        n.wait()

        def comp_row(e, _):
            def comp_col(q, _):
                v = rows[e, pl.ds(q * L, L)]
                w = nbuf[e, pl.ds(q * L, L)]
                mbuf[e, pl.ds(q * L, L)] = v + v * w
                return 0

            lax.fori_loop(0, D // L, comp_col, 0, unroll=True)
            return 0

        lax.fori_loop(0, CHUNK, comp_row, 0)

        pltpu.sync_copy(mbuf, agg.at[didx], add=True)
        return 0

    lax.fori_loop(0, NCHUNK, chunk_body, 0)
    plsc.subcore_barrier()

    for r in range(ROWS_PER_TILE // ZROWS):
        rb = abase + r * ZROWS
        pltpu.sync_copy(agg.at[pl.ds(rb, ZROWS), :], out_hbm.at[c, pl.ds(rb, ZROWS), :])


def _finish_body(p0, p1, cnt, w, bb, o):
    agg = p0[...] + p1[...]
    scale = lax.rsqrt(jnp.maximum(cnt[...], 1.0))
    o[...] = jnp.dot(agg * scale, w[...], preferred_element_type=jnp.float32) + bb[...]


_FIN_BLOCK = 2000


def _finish(p0, p1, cnt2d, W, b2d):
    return pl.pallas_call(
        _finish_body,
        out_shape=jax.ShapeDtypeStruct((N, D), jnp.float32),
        grid=(N // _FIN_BLOCK,),
        in_specs=[
            pl.BlockSpec((_FIN_BLOCK, D), lambda i: (i, 0)),
            pl.BlockSpec((_FIN_BLOCK, D), lambda i: (i, 0)),
            pl.BlockSpec((_FIN_BLOCK, 1), lambda i: (i, 0)),
            pl.BlockSpec((D, D), lambda i: (0, 0)),
            pl.BlockSpec((1, D), lambda i: (0, 0)),
        ],
        out_specs=pl.BlockSpec((_FIN_BLOCK, D), lambda i: (i, 0)),
    )(p0, p1, cnt2d, W, b2d)


def kernel(feat, edge_index, edge_noise, W, b):
    src = edge_index[0]
    dst = edge_index[1]
    counts = _degrees(src, dst)
    norm_src = lax.rsqrt(jnp.maximum(counts[:N], 1.0))
    h = feat * norm_src[:, None]
    parts = _aggregate(h, src, dst, edge_noise)
    cnt_dst = counts[NPAD:NPAD + N].reshape(N, 1)
    return _finish(parts[0, :N], parts[1, :N], cnt_dst, W, b.reshape(1, D))


# speedup vs baseline: 7.2630x; 1.9219x over previous
import functools

import jax
import jax.numpy as jnp
from jax import lax
from jax.experimental import pallas as pl
from jax.experimental.pallas import tpu as pltpu
from jax.experimental.pallas import tpu_sc as plsc

N = 10000
E = 320000
D = 128
L = 16
NC = 2
NS = 16
NW = NC * NS
NPAD = 10240

E_PER_TILE_DEG = E // NS
E_PER_TILE = E // NW
CHUNK = 80
NCHUNK = E_PER_TILE // CHUNK
ROWS_PER_TILE = NPAD // NS
ZROWS = 128

_mesh = plsc.VectorSubcoreMesh(core_axis_name="c", subcore_axis_name="s")
_sc_params = pltpu.CompilerParams(needs_layout_passes=False)


@functools.partial(
    pl.kernel,
    out_type=jax.ShapeDtypeStruct((2 * NPAD,), jnp.float32),
    mesh=_mesh,
    scratch_types=[
        pltpu.VMEM((E_PER_TILE_DEG,), jnp.int32),
        pltpu.VMEM((NPAD,), jnp.float32),
        pltpu.VMEM((NPAD // NS,), jnp.float32),
        pltpu.VMEM((NPAD // NS,), jnp.float32),
        pltpu.VMEM_SHARED((NS, NPAD), jnp.float32),
    ],
    compiler_params=_sc_params,
)
def _degrees(src_hbm, dst_hbm, out_hbm, idx_v, acc, tbuf, sbuf, shared):
    c = lax.axis_index("c")
    s = lax.axis_index("s")
    base = s * E_PER_TILE_DEG

    @pl.when(c == 0)
    def _():
        pltpu.sync_copy(src_hbm.at[pl.ds(base, E_PER_TILE_DEG)], idx_v)

    @pl.when(c == 1)
    def _():
        pltpu.sync_copy(dst_hbm.at[pl.ds(base, E_PER_TILE_DEG)], idx_v)

    zeros16 = jnp.zeros((L,), jnp.float32)
    ones16 = jnp.ones((L,), jnp.float32)

    def zero_body(i, _):
        acc[pl.ds(i * L, L)] = zeros16
        return 0

    lax.fori_loop(0, NPAD // L, zero_body, 0)

    def count_body(i, _):
        iv = idx_v[pl.ds(i * L, L)]
        plsc.addupdate_scatter(acc, [iv], ones16)
        return 0

    lax.fori_loop(0, E_PER_TILE_DEG // L, count_body, 0)

    pltpu.sync_copy(acc, shared.at[s])
    plsc.subcore_barrier()

    seg = NPAD // NS
    rbase = s * seg

    def zred(i, _):
        sbuf[pl.ds(i * L, L)] = zeros16
        return 0

    lax.fori_loop(0, seg // L, zred, 0)

    for t in range(NS):
        pltpu.sync_copy(shared.at[t, pl.ds(rbase, seg)], tbuf)

        def radd(i, _):
            sbuf[pl.ds(i * L, L)] = sbuf[pl.ds(i * L, L)] + tbuf[pl.ds(i * L, L)]
            return 0

        lax.fori_loop(0, seg // L, radd, 0)

    pltpu.sync_copy(sbuf, out_hbm.at[pl.ds(c * NPAD + rbase, seg)])


@functools.partial(
    pl.kernel,
    out_type=jax.ShapeDtypeStruct((NC, NPAD, D), jnp.float32),
    mesh=_mesh,
    scratch_types=[
        pltpu.VMEM((2, CHUNK), jnp.int32),
        pltpu.VMEM((2, CHUNK), jnp.int32),
        pltpu.VMEM((2, CHUNK), jnp.int32),
        pltpu.VMEM((2, CHUNK), jnp.int32),
        pltpu.VMEM((CHUNK, D), jnp.float32),
        pltpu.VMEM((CHUNK, D), jnp.float32),
        pltpu.VMEM((CHUNK, D), jnp.float32),
        pltpu.VMEM((CHUNK, D), jnp.float32),
        pltpu.VMEM_SHARED((NPAD, D), jnp.float32),
        pltpu.SemaphoreType.DMA,
        pltpu.SemaphoreType.DMA,
        pltpu.SemaphoreType.DMA,
        pltpu.SemaphoreType.DMA,
        pltpu.SemaphoreType.DMA,
        pltpu.SemaphoreType.DMA,
        pltpu.SemaphoreType.DMA,
        pltpu.SemaphoreType.DMA,
    ],
    compiler_params=_sc_params,
)
def _aggregate(h_hbm, eidx_hbm, noise_hbm, out_hbm,
               ib0, ib1, ib2, ib3, rows_a, rows_b, nbuf_a, nbuf_b, agg,
               si0, si1, si2, si3, sga, sgb, sna, snb):
    c = lax.axis_index("c")
    s = lax.axis_index("s")
    wid = c * NS + s
    ebase = wid * E_PER_TILE

    ibufs = [ib0, ib1, ib2, ib3]
    isems = [si0, si1, si2, si3]
    zeros16 = jnp.zeros((L,), jnp.float32)

    def zrow(r, _):
        def zcol(q, _):
            rows_a[r, pl.ds(q * L, L)] = zeros16
            return 0
        lax.fori_loop(0, D // L, zcol, 0, unroll=True)
        return 0

    lax.fori_loop(0, CHUNK, zrow, 0)

    abase = s * ROWS_PER_TILE
    for r in range(ROWS_PER_TILE // CHUNK):
        pltpu.sync_copy(rows_a, agg.at[pl.ds(abase + r * CHUNK, CHUNK), :])
    plsc.subcore_barrier()

    def fetch_idx(k, slot):
        pltpu.async_copy(eidx_hbm.at[wid, k], ibufs[slot], isems[slot])

    def wait_idx(k, slot):
        pltpu.make_async_copy(eidx_hbm.at[wid, k], ibufs[slot], isems[slot]).wait()

    def fetch_data(k, slot, rows_v, nbuf_v, sg, sn):
        pltpu.async_copy(h_hbm.at[ibufs[slot].at[0]], rows_v, sg)
        pltpu.async_copy(noise_hbm.at[pl.ds(ebase + k * CHUNK, CHUNK), :], nbuf_v, sn)

    def wait_data(k, slot, rows_v, nbuf_v, sg, sn):
        pltpu.make_async_copy(h_hbm.at[ibufs[slot].at[0]], rows_v, sg).wait()
        pltpu.make_async_copy(
            noise_hbm.at[pl.ds(ebase + k * CHUNK, CHUNK), :], nbuf_v, sn).wait()

    def compute(rows_v, nbuf_v):
        def comp_row(e, _):
            def comp_col(q, _):
                v = rows_v[e, pl.ds(q * L, L)]
                w = nbuf_v[e, pl.ds(q * L, L)]
                rows_v[e, pl.ds(q * L, L)] = v + v * w
                return 0
            lax.fori_loop(0, D // L, comp_col, 0, unroll=True)
            return 0
        lax.fori_loop(0, CHUNK, comp_row, 0)

    def scatter(slot, rows_v):
        pltpu.sync_copy(rows_v, agg.at[ibufs[slot].at[1]], add=True)

    pltpu.sync_copy(eidx_hbm.at[wid, 0], ib0)
    pltpu.sync_copy(eidx_hbm.at[wid, 1], ib1)
    fetch_data(0, 0, rows_a, nbuf_a, sga, sna)
    fetch_data(1, 1, rows_b, nbuf_b, sgb, snb)

    bufs = ((rows_a, nbuf_a, sga, sna), (rows_b, nbuf_b, sgb, snb))

    def step(k, slot, fetch_next):
        rows_v, nbuf_v, sg, sn = bufs[slot % 2]
        nslot = (slot + 2) % 4
        if fetch_next:
            fetch_idx(k + 2, nslot)
        wait_data(k, slot, rows_v, nbuf_v, sg, sn)
        compute(rows_v, nbuf_v)
        scatter(slot, rows_v)
        if fetch_next:
            wait_idx(k + 2, nslot)
            fetch_data(k + 2, nslot, rows_v, nbuf_v, sg, sn)

    def quad(j, _):
        k0 = j * 4
        for d in range(4):
            step(k0 + d, d, True)
        return 0

    lax.fori_loop(0, (NCHUNK - 5) // 4, quad, 0)

    base = NCHUNK - 5
    step(base + 0, 0, True)
    step(base + 1, 1, True)
    step(base + 2, 2, True)
    step(base + 3, 3, False)
    step(base + 4, 0, False)

    plsc.subcore_barrier()
    for r in range(ROWS_PER_TILE // CHUNK):
        rb = abase + r * CHUNK
        pltpu.sync_copy(agg.at[pl.ds(rb, CHUNK), :], out_hbm.at[c, pl.ds(rb, CHUNK), :])


def _finish_body(p0, p1, cnt, w, bb, o):
    agg = p0[...] + p1[...]
    scale = lax.rsqrt(jnp.maximum(cnt[...], 1.0))
    o[...] = jnp.dot(agg * scale, w[...], preferred_element_type=jnp.float32) + bb[...]


_FIN_BLOCK = 2000


def _finish(p0, p1, cnt2d, W, b2d):
    return pl.pallas_call(
        _finish_body,
        out_shape=jax.ShapeDtypeStruct((N, D), jnp.float32),
        grid=(N // _FIN_BLOCK,),
        in_specs=[
            pl.BlockSpec((_FIN_BLOCK, D), lambda i: (i, 0)),
            pl.BlockSpec((_FIN_BLOCK, D), lambda i: (i, 0)),
            pl.BlockSpec((_FIN_BLOCK, 1), lambda i: (i, 0)),
            pl.BlockSpec((D, D), lambda i: (0, 0)),
            pl.BlockSpec((1, D), lambda i: (0, 0)),
        ],
        out_specs=pl.BlockSpec((_FIN_BLOCK, D), lambda i: (i, 0)),
    )(p0, p1, cnt2d, W, b2d)


def kernel(feat, edge_index, edge_noise, W, b):
    src = edge_index[0]
    dst = edge_index[1]
    counts = _degrees(src, dst)
    norm_src = lax.rsqrt(jnp.maximum(counts[:N], 1.0))
    h = feat * norm_src[:, None]
    eidx = edge_index.reshape(2, NW, NCHUNK, CHUNK).transpose(1, 2, 0, 3)
    parts = _aggregate(h, eidx, edge_noise)
    cnt_dst = counts[NPAD:NPAD + N].reshape(N, 1)
    return _finish(parts[0, :N], parts[1, :N], cnt_dst, W, b.reshape(1, D))


# speedup vs baseline: 7.5915x; 1.0452x over previous
import functools

import jax
import jax.numpy as jnp
from jax import lax
from jax.experimental import pallas as pl
from jax.experimental.pallas import tpu as pltpu
from jax.experimental.pallas import tpu_sc as plsc

N = 10000
E = 320000
D = 128
L = 16
NC = 2
NS = 16
NW = NC * NS
NPAD = 10240

E_PER_TILE_DEG = E // NS
E_PER_TILE = E // NW
CHUNK = 80
NCHUNK = E_PER_TILE // CHUNK
ROWS_PER_TILE = NPAD // NS
ZROWS = 128

_mesh = plsc.VectorSubcoreMesh(core_axis_name="c", subcore_axis_name="s")
_sc_params = pltpu.CompilerParams(needs_layout_passes=False, use_tc_tiling_on_sc=False)


@functools.partial(
    pl.kernel,
    out_type=jax.ShapeDtypeStruct((2 * NPAD,), jnp.float32),
    mesh=_mesh,
    scratch_types=[
        pltpu.VMEM((E_PER_TILE_DEG,), jnp.int32),
        pltpu.VMEM((NPAD,), jnp.float32),
        pltpu.VMEM((NPAD // NS,), jnp.float32),
        pltpu.VMEM((NPAD // NS,), jnp.float32),
        pltpu.VMEM_SHARED((NS, NPAD), jnp.float32),
    ],
    compiler_params=_sc_params,
)
def _degrees(src_hbm, dst_hbm, out_hbm, idx_v, acc, tbuf, sbuf, shared):
    c = lax.axis_index("c")
    s = lax.axis_index("s")
    base = s * E_PER_TILE_DEG

    @pl.when(c == 0)
    def _():
        pltpu.sync_copy(src_hbm.at[pl.ds(base, E_PER_TILE_DEG)], idx_v)

    @pl.when(c == 1)
    def _():
        pltpu.sync_copy(dst_hbm.at[pl.ds(base, E_PER_TILE_DEG)], idx_v)

    zeros16 = jnp.zeros((L,), jnp.float32)
    ones16 = jnp.ones((L,), jnp.float32)

    def zero_body(i, _):
        acc[pl.ds(i * L, L)] = zeros16
        return 0

    lax.fori_loop(0, NPAD // L, zero_body, 0)

    def count_body(i, _):
        iv = idx_v[pl.ds(i * L, L)]
        plsc.addupdate_scatter(acc, [iv], ones16)
        return 0

    lax.fori_loop(0, E_PER_TILE_DEG // L, count_body, 0)

    pltpu.sync_copy(acc, shared.at[s])
    plsc.subcore_barrier()

    seg = NPAD // NS
    rbase = s * seg

    def zred(i, _):
        sbuf[pl.ds(i * L, L)] = zeros16
        return 0

    lax.fori_loop(0, seg // L, zred, 0)

    for t in range(NS):
        pltpu.sync_copy(shared.at[t, pl.ds(rbase, seg)], tbuf)

        def radd(i, _):
            sbuf[pl.ds(i * L, L)] = sbuf[pl.ds(i * L, L)] + tbuf[pl.ds(i * L, L)]
            return 0

        lax.fori_loop(0, seg // L, radd, 0)

    pltpu.sync_copy(sbuf, out_hbm.at[pl.ds(c * NPAD + rbase, seg)])


@functools.partial(
    pl.kernel,
    out_type=jax.ShapeDtypeStruct((NC, NPAD, D), jnp.float32),
    mesh=_mesh,
    scratch_types=[
        pltpu.VMEM((2, CHUNK), jnp.int32),
        pltpu.VMEM((2, CHUNK), jnp.int32),
        pltpu.VMEM((2, CHUNK), jnp.int32),
        pltpu.VMEM((2, CHUNK), jnp.int32),
        pltpu.VMEM((CHUNK, D // 2), jnp.int32),
        pltpu.VMEM((CHUNK, D // 2), jnp.int32),
        pltpu.VMEM((CHUNK, D), jnp.float32),
        pltpu.VMEM((CHUNK, D), jnp.float32),
        pltpu.VMEM_SHARED((NPAD, D), jnp.float32),
        pltpu.SemaphoreType.DMA,
        pltpu.SemaphoreType.DMA,
        pltpu.SemaphoreType.DMA,
        pltpu.SemaphoreType.DMA,
        pltpu.SemaphoreType.DMA,
        pltpu.SemaphoreType.DMA,
        pltpu.SemaphoreType.DMA,
        pltpu.SemaphoreType.DMA,
    ],
    compiler_params=_sc_params,
)
def _aggregate(h_hbm, eidx_hbm, noise_hbm, out_hbm,
               ib0, ib1, ib2, ib3, rows_a, rows_b, nbuf_a, nbuf_b, agg,
               si0, si1, si2, si3, sga, sgb, sna, snb):
    c = lax.axis_index("c")
    s = lax.axis_index("s")
    wid = c * NS + s
    ebase = wid * E_PER_TILE

    ibufs = [ib0, ib1, ib2, ib3]
    isems = [si0, si1, si2, si3]
    zeros16 = jnp.zeros((L,), jnp.float32)

    def zrow(r, _):
        def zcol(q, _):
            nbuf_a[r, pl.ds(q * L, L)] = zeros16
            return 0
        lax.fori_loop(0, D // L, zcol, 0, unroll=True)
        return 0

    lax.fori_loop(0, CHUNK, zrow, 0)

    abase = s * ROWS_PER_TILE
    for r in range(ROWS_PER_TILE // CHUNK):
        pltpu.sync_copy(nbuf_a, agg.at[pl.ds(abase + r * CHUNK, CHUNK), :])
    plsc.subcore_barrier()

    def fetch_idx(k, slot):
        pltpu.async_copy(eidx_hbm.at[wid, k], ibufs[slot], isems[slot])

    def wait_idx(k, slot):
        pltpu.make_async_copy(eidx_hbm.at[wid, k], ibufs[slot], isems[slot]).wait()

    def fetch_data(k, slot, rows_v, nbuf_v, sg, sn):
        pltpu.async_copy(h_hbm.at[ibufs[slot].at[0]], rows_v, sg)
        pltpu.async_copy(noise_hbm.at[pl.ds(ebase + k * CHUNK, CHUNK), :], nbuf_v, sn)

    def wait_data(k, slot, rows_v, nbuf_v, sg, sn):
        pltpu.make_async_copy(h_hbm.at[ibufs[slot].at[0]], rows_v, sg).wait()
        pltpu.make_async_copy(
            noise_hbm.at[pl.ds(ebase + k * CHUNK, CHUNK), :], nbuf_v, sn).wait()

    def compute(rows_v, nbuf_v):
        @plsc.parallel_loop(0, CHUNK, 1, unroll=2)
        def _(e):
            for g in range(D // (2 * L)):
                hw = rows_v[e, pl.ds(L * g, L)]
                hb = plsc.bitcast(hw, jnp.bfloat16)
                va, vb = plsc.unpack(hb, format=plsc.PackFormat.INTERLEAVED)
                wa = nbuf_v[e, pl.ds(2 * L * g, L)]
                wb = nbuf_v[e, pl.ds(2 * L * g + L, L)]
                nbuf_v[e, pl.ds(2 * L * g, L)] = va + va * wa
                nbuf_v[e, pl.ds(2 * L * g + L, L)] = vb + vb * wb

    def scatter(slot, nbuf_v):
        pltpu.sync_copy(nbuf_v, agg.at[ibufs[slot].at[1]], add=True)

    pltpu.sync_copy(eidx_hbm.at[wid, 0], ib0)
    pltpu.sync_copy(eidx_hbm.at[wid, 1], ib1)
    fetch_data(0, 0, rows_a, nbuf_a, sga, sna)
    fetch_data(1, 1, rows_b, nbuf_b, sgb, snb)

    bufs = ((rows_a, nbuf_a, sga, sna), (rows_b, nbuf_b, sgb, snb))

    def step(k, slot, fetch_next):
        rows_v, nbuf_v, sg, sn = bufs[slot % 2]
        nslot = (slot + 2) % 4
        if fetch_next:
            fetch_idx(k + 2, nslot)
        wait_data(k, slot, rows_v, nbuf_v, sg, sn)
        compute(rows_v, nbuf_v)
        scatter(slot, nbuf_v)
        if fetch_next:
            wait_idx(k + 2, nslot)
            fetch_data(k + 2, nslot, rows_v, nbuf_v, sg, sn)

    def quad(j, _):
        k0 = j * 4
        for d in range(4):
            step(k0 + d, d, True)
        return 0

    lax.fori_loop(0, (NCHUNK - 5) // 4, quad, 0)

    base = NCHUNK - 5
    step(base + 0, 0, True)
    step(base + 1, 1, True)
    step(base + 2, 2, True)
    step(base + 3, 3, False)
    step(base + 4, 0, False)

    plsc.subcore_barrier()
    for r in range(ROWS_PER_TILE // CHUNK):
        rb = abase + r * CHUNK
        pltpu.sync_copy(agg.at[pl.ds(rb, CHUNK), :], out_hbm.at[c, pl.ds(rb, CHUNK), :])


def _finish_body(p0, p1, cnt, w, bb, o):
    agg = p0[...] + p1[...]
    scale = lax.rsqrt(jnp.maximum(cnt[...], 1.0))
    o[...] = jnp.dot(agg * scale, w[...], preferred_element_type=jnp.float32) + bb[...]


_FIN_BLOCK = 2000


def _finish(p0, p1, cnt2d, W, b2d):
    return pl.pallas_call(
        _finish_body,
        out_shape=jax.ShapeDtypeStruct((N, D), jnp.float32),
        grid=(N // _FIN_BLOCK,),
        in_specs=[
            pl.BlockSpec((_FIN_BLOCK, D), lambda i: (i, 0)),
            pl.BlockSpec((_FIN_BLOCK, D), lambda i: (i, 0)),
            pl.BlockSpec((_FIN_BLOCK, 1), lambda i: (i, 0)),
            pl.BlockSpec((D, D), lambda i: (0, 0)),
            pl.BlockSpec((1, D), lambda i: (0, 0)),
        ],
        out_specs=pl.BlockSpec((_FIN_BLOCK, D), lambda i: (i, 0)),
    )(p0, p1, cnt2d, W, b2d)


def kernel(feat, edge_index, edge_noise, W, b):
    src = edge_index[0]
    dst = edge_index[1]
    counts = _degrees(src, dst)
    norm_src = lax.rsqrt(jnp.maximum(counts[:N], 1.0))
    h = feat * norm_src[:, None]
    h_bf = (h.reshape(N, D // 32, 2, 16).transpose(0, 1, 3, 2)
             .reshape(N, D).astype(jnp.bfloat16))
    h_pk = lax.bitcast_convert_type(h_bf.reshape(N, D // 2, 2), jnp.int32)
    eidx = edge_index.reshape(2, NW, NCHUNK, CHUNK).transpose(1, 2, 0, 3)
    parts = _aggregate(h_pk, eidx, edge_noise)
    cnt_dst = counts[NPAD:NPAD + N].reshape(N, 1)
    return _finish(parts[0, :N], parts[1, :N], cnt_dst, W, b.reshape(1, D))
